# initial kernel scaffold (unmeasured)
import jax
import jax.numpy as jnp
from jax import lax
from jax.experimental import pallas as pl
from jax.experimental.pallas import tpu as pltpu

N_DEV = 8


def kernel(A, B):
    m_per, k = A.shape
    _, n = B.shape

    A = A.astype(jnp.bfloat16)
    B = B.astype(jnp.bfloat16)

    def body(a_ref, b_ref, out_ref, ag_ref, send_sems, recv_sems):
        my = lax.axis_index("i")
        left = lax.rem(my - 1 + N_DEV, N_DEV)
        right = lax.rem(my + 1, N_DEV)

        barrier_sem = pltpu.get_barrier_semaphore()
        for nbr in (left, right):
            pl.semaphore_signal(
                barrier_sem, inc=1,
                device_id=(nbr,), device_id_type=pl.DeviceIdType.MESH,
            )
        pl.semaphore_wait(barrier_sem, 2)

        ag_ref[my] = a_ref[...]
        out_ref[pl.ds(my * m_per, m_per), :] = jnp.dot(
            a_ref[...], b_ref[...], preferred_element_type=jnp.float32
        ).astype(out_ref.dtype)

        for h in range(N_DEV - 1):
            send_chunk = lax.rem(my - h + N_DEV, N_DEV)
            recv_chunk = lax.rem(my - h - 1 + N_DEV, N_DEV)
            rdma = pltpu.make_async_remote_copy(
                src_ref=ag_ref.at[send_chunk],
                dst_ref=ag_ref.at[send_chunk],
                send_sem=send_sems.at[h],
                recv_sem=recv_sems.at[h],
                device_id=(right,),
                device_id_type=pl.DeviceIdType.MESH,
            )
            rdma.start()
            rdma.wait()
            out_ref[pl.ds(recv_chunk * m_per, m_per), :] = jnp.dot(
                ag_ref[recv_chunk], b_ref[...],
                preferred_element_type=jnp.float32,
            ).astype(out_ref.dtype)

    return pl.pallas_call(
        body,
        out_shape=jax.ShapeDtypeStruct((N_DEV * m_per, n), jnp.bfloat16),
        in_specs=[
            pl.BlockSpec(memory_space=pltpu.VMEM),
            pl.BlockSpec(memory_space=pltpu.VMEM),
        ],
        out_specs=pl.BlockSpec(memory_space=pltpu.VMEM),
        scratch_shapes=[
            pltpu.VMEM((N_DEV, m_per, k), jnp.bfloat16),
            pltpu.SemaphoreType.DMA((N_DEV - 1,)),
            pltpu.SemaphoreType.DMA((N_DEV - 1,)),
        ],
        compiler_params=pltpu.CompilerParams(collective_id=0),
    )(A, B)


# baseline (device time: 272899 ns/iter reference)
import jax
import jax.numpy as jnp
from jax import lax
from jax.experimental import pallas as pl
from jax.experimental.pallas import tpu as pltpu

N_DEV = 8


def kernel(A, B):
    m_per, k = A.shape
    _, n = B.shape

    A = A.astype(jnp.bfloat16)
    B = B.astype(jnp.bfloat16)

    def body(a_ref, b_ref, out_ref, ag_ref, stage_ref, send_sems, recv_sems,
             copy_sem):
        my = lax.axis_index("i")
        left = lax.rem(my - 1 + N_DEV, N_DEV)
        right = lax.rem(my + 1, N_DEV)

        def compute_chunk(chunk):
            stage_ref[...] = jnp.dot(
                ag_ref[chunk], b_ref[...], preferred_element_type=jnp.float32
            ).astype(stage_ref.dtype)
            copy = pltpu.make_async_copy(
                stage_ref, out_ref.at[pl.ds(chunk * m_per, m_per)], copy_sem
            )
            copy.start()
            copy.wait()

        barrier_sem = pltpu.get_barrier_semaphore()
        for nbr in (left, right):
            pl.semaphore_signal(
                barrier_sem, inc=1,
                device_id=(nbr,), device_id_type=pl.DeviceIdType.MESH,
            )
        pl.semaphore_wait(barrier_sem, 2)

        ag_ref[my] = a_ref[...]
        compute_chunk(my)

        for h in range(N_DEV - 1):
            send_chunk = lax.rem(my - h + N_DEV, N_DEV)
            recv_chunk = lax.rem(my - h - 1 + N_DEV, N_DEV)
            rdma = pltpu.make_async_remote_copy(
                src_ref=ag_ref.at[send_chunk],
                dst_ref=ag_ref.at[send_chunk],
                send_sem=send_sems.at[h],
                recv_sem=recv_sems.at[h],
                device_id=(right,),
                device_id_type=pl.DeviceIdType.MESH,
            )
            rdma.start()
            rdma.wait()
            compute_chunk(recv_chunk)

    return pl.pallas_call(
        body,
        out_shape=jax.ShapeDtypeStruct((N_DEV * m_per, n), jnp.bfloat16),
        in_specs=[
            pl.BlockSpec(memory_space=pltpu.VMEM),
            pl.BlockSpec(memory_space=pltpu.VMEM),
        ],
        out_specs=pl.BlockSpec(memory_space=pltpu.MemorySpace.HBM),
        scratch_shapes=[
            pltpu.VMEM((N_DEV, m_per, k), jnp.bfloat16),
            pltpu.VMEM((m_per, n), jnp.bfloat16),
            pltpu.SemaphoreType.DMA((N_DEV - 1,)),
            pltpu.SemaphoreType.DMA((N_DEV - 1,)),
            pltpu.SemaphoreType.DMA,
        ],
        compiler_params=pltpu.CompilerParams(collective_id=0),
    )(A, B)


# device time: 227727 ns/iter; 1.1984x vs baseline; 1.1984x over previous
import jax
import jax.numpy as jnp
from jax import lax
from jax.experimental import pallas as pl
from jax.experimental.pallas import tpu as pltpu

N_DEV = 8


def kernel(A, B):
    m_per, k = A.shape
    _, n = B.shape

    A = A.astype(jnp.bfloat16)
    B = B.astype(jnp.bfloat16)

    def body(a_ref, b_ref, out_ref, ag_ref, stage_ref, send_sems, recv_sems,
             copy_sems):
        my = lax.axis_index("i")
        left = lax.rem(my - 1 + N_DEV, N_DEV)
        right = lax.rem(my + 1, N_DEV)

        copies = {}

        def compute_iter(i):
            chunk = lax.rem(my - i + N_DEV, N_DEV)
            slot = i % 2
            if i >= 2:
                copies[i - 2].wait()
            stage_ref[slot] = jnp.dot(
                ag_ref[chunk], b_ref[...], preferred_element_type=jnp.float32
            ).astype(stage_ref.dtype)
            copies[i] = pltpu.make_async_copy(
                stage_ref.at[slot],
                out_ref.at[pl.ds(chunk * m_per, m_per)],
                copy_sems.at[slot],
            )
            copies[i].start()

        barrier_sem = pltpu.get_barrier_semaphore()
        for nbr in (left, right):
            pl.semaphore_signal(
                barrier_sem, inc=1,
                device_id=(nbr,), device_id_type=pl.DeviceIdType.MESH,
            )
        pl.semaphore_wait(barrier_sem, 2)

        ag_ref[my] = a_ref[...]

        def make_rdma(h):
            send_chunk = lax.rem(my - h + N_DEV, N_DEV)
            return pltpu.make_async_remote_copy(
                src_ref=ag_ref.at[send_chunk],
                dst_ref=ag_ref.at[send_chunk],
                send_sem=send_sems.at[h],
                recv_sem=recv_sems.at[h],
                device_id=(right,),
                device_id_type=pl.DeviceIdType.MESH,
            )

        rdmas = [make_rdma(h) for h in range(N_DEV - 1)]

        rdmas[0].start()
        compute_iter(0)
        for h in range(1, N_DEV - 1):
            rdmas[h - 1].wait_recv()
            rdmas[h].start()
            compute_iter(h)
        rdmas[N_DEV - 2].wait_recv()
        compute_iter(N_DEV - 1)

        for h in range(N_DEV - 1):
            rdmas[h].wait_send()
        copies[N_DEV - 2].wait()
        copies[N_DEV - 1].wait()

    return pl.pallas_call(
        body,
        out_shape=jax.ShapeDtypeStruct((N_DEV * m_per, n), jnp.bfloat16),
        in_specs=[
            pl.BlockSpec(memory_space=pltpu.VMEM),
            pl.BlockSpec(memory_space=pltpu.VMEM),
        ],
        out_specs=pl.BlockSpec(memory_space=pltpu.MemorySpace.HBM),
        scratch_shapes=[
            pltpu.VMEM((N_DEV, m_per, k), jnp.bfloat16),
            pltpu.VMEM((2, m_per, n), jnp.bfloat16),
            pltpu.SemaphoreType.DMA((N_DEV - 1,)),
            pltpu.SemaphoreType.DMA((N_DEV - 1,)),
            pltpu.SemaphoreType.DMA((2,)),
        ],
        compiler_params=pltpu.CompilerParams(collective_id=0),
    )(A, B)


# device time: 146957 ns/iter; 1.8570x vs baseline; 1.5496x over previous
import jax
import jax.numpy as jnp
from jax import lax
from jax.experimental import pallas as pl
from jax.experimental.pallas import tpu as pltpu

N_DEV = 8
R_HOPS = 4
L_HOPS = 3


def kernel(A, B):
    m_per, k = A.shape
    _, n = B.shape

    A = A.astype(jnp.bfloat16)
    B = B.astype(jnp.bfloat16)

    def body(a_ref, b_ref, out_ref, ag_ref, stage_ref,
             send_sems_r, recv_sems_r, send_sems_l, recv_sems_l, copy_sems):
        my = lax.axis_index("i")
        left = lax.rem(my - 1 + N_DEV, N_DEV)
        right = lax.rem(my + 1, N_DEV)

        copies = {}
        it = [0]

        def compute_chunk(chunk):
            i = it[0]
            it[0] += 1
            slot = i % 2
            if i >= 2:
                copies[i - 2].wait()
            stage_ref[slot] = jnp.dot(
                ag_ref[chunk], b_ref[...], preferred_element_type=jnp.float32
            ).astype(stage_ref.dtype)
            copies[i] = pltpu.make_async_copy(
                stage_ref.at[slot],
                out_ref.at[pl.ds(chunk * m_per, m_per)],
                copy_sems.at[slot],
            )
            copies[i].start()

        def make_rdma(chunk, sems, h, target):
            return pltpu.make_async_remote_copy(
                src_ref=ag_ref.at[chunk],
                dst_ref=ag_ref.at[chunk],
                send_sem=sems[0].at[h],
                recv_sem=sems[1].at[h],
                device_id=(target,),
                device_id_type=pl.DeviceIdType.MESH,
            )

        barrier_sem = pltpu.get_barrier_semaphore()
        for nbr in (left, right):
            pl.semaphore_signal(
                barrier_sem, inc=1,
                device_id=(nbr,), device_id_type=pl.DeviceIdType.MESH,
            )
        pl.semaphore_wait(barrier_sem, 2)

        ag_ref[my] = a_ref[...]

        sems_r = (send_sems_r, recv_sems_r)
        sems_l = (send_sems_l, recv_sems_l)

        rdmas_r = [make_rdma(my, sems_r, 0, right)]
        rdmas_l = [make_rdma(my, sems_l, 0, left)]
        rdmas_r[0].start()
        rdmas_l[0].start()
        compute_chunk(my)

        for r in range(1, R_HOPS + 1):
            rchunk = lax.rem(my - r + N_DEV, N_DEV)
            rdmas_r[r - 1].wait_recv()
            if r < R_HOPS:
                rdmas_r.append(make_rdma(rchunk, sems_r, r, right))
                rdmas_r[r].start()
            lchunk = None
            if r <= L_HOPS:
                lchunk = lax.rem(my + r, N_DEV)
                rdmas_l[r - 1].wait_recv()
                if r < L_HOPS:
                    rdmas_l.append(make_rdma(lchunk, sems_l, r, left))
                    rdmas_l[r].start()
            compute_chunk(rchunk)
            if lchunk is not None:
                compute_chunk(lchunk)

        for rd in rdmas_r + rdmas_l:
            rd.wait_send()
        copies[N_DEV - 2].wait()
        copies[N_DEV - 1].wait()

    return pl.pallas_call(
        body,
        out_shape=jax.ShapeDtypeStruct((N_DEV * m_per, n), jnp.bfloat16),
        in_specs=[
            pl.BlockSpec(memory_space=pltpu.VMEM),
            pl.BlockSpec(memory_space=pltpu.VMEM),
        ],
        out_specs=pl.BlockSpec(memory_space=pltpu.MemorySpace.HBM),
        scratch_shapes=[
            pltpu.VMEM((N_DEV, m_per, k), jnp.bfloat16),
            pltpu.VMEM((2, m_per, n), jnp.bfloat16),
            pltpu.SemaphoreType.DMA((R_HOPS,)),
            pltpu.SemaphoreType.DMA((R_HOPS,)),
            pltpu.SemaphoreType.DMA((L_HOPS,)),
            pltpu.SemaphoreType.DMA((L_HOPS,)),
            pltpu.SemaphoreType.DMA((2,)),
        ],
        compiler_params=pltpu.CompilerParams(collective_id=0),
    )(A, B)


# device time: 120812 ns/iter; 2.2589x vs baseline; 1.2164x over previous
import functools

import jax
import jax.numpy as jnp
from jax import lax
from jax.experimental import pallas as pl
from jax.experimental.pallas import tpu as pltpu

N_DEV = 8


def kernel(A, B):
    m_per, k = A.shape
    _, n = B.shape

    A = A.astype(jnp.bfloat16)
    B = B.astype(jnp.bfloat16)

    def body(a_ref, b_ref, out_ref, ag_ref, stage_ref, send_sems, recv_sems,
             copy_sems):
        my = lax.axis_index("i")

        z = my // 4
        p2 = lax.rem(my, 4)
        y = p2 // 2
        x = ((p2 == 1) | (p2 == 2)).astype(my.dtype)

        def pos(xx, yy, zz):
            return zz * 4 + xx + yy * (3 - 2 * xx)

        nbr = [pos(1 - x, y, z), pos(x, 1 - y, z), pos(x, y, 1 - z)]
        chunk_xy = pos(1 - x, 1 - y, z)
        chunk_xz = pos(1 - x, y, 1 - z)
        chunk_yz = pos(x, 1 - y, 1 - z)
        chunk_xyz = pos(1 - x, 1 - y, 1 - z)

        copies = {}
        it = [0]

        def compute_chunk(chunk):
            i = it[0]
            it[0] += 1
            slot = i % 2
            if i >= 2:
                copies[i - 2].wait()
            stage_ref[slot] = jnp.dot(
                ag_ref[chunk], b_ref[...], preferred_element_type=jnp.float32
            ).astype(stage_ref.dtype)
            copies[i] = pltpu.make_async_copy(
                stage_ref.at[slot],
                out_ref.at[pl.ds(chunk * m_per, m_per)],
                copy_sems.at[slot],
            )
            copies[i].start()

        def make_rdma(chunk, phase, link):
            return pltpu.make_async_remote_copy(
                src_ref=ag_ref.at[chunk],
                dst_ref=ag_ref.at[chunk],
                send_sem=send_sems.at[phase, link],
                recv_sem=recv_sems.at[phase, link],
                device_id=(nbr[link],),
                device_id_type=pl.DeviceIdType.MESH,
            )

        barrier_sem = pltpu.get_barrier_semaphore()
        for l in range(3):
            pl.semaphore_signal(
                barrier_sem, inc=1,
                device_id=(nbr[l],), device_id_type=pl.DeviceIdType.MESH,
            )
        pl.semaphore_wait(barrier_sem, 3)

        ag_ref[my] = a_ref[...]

        p1 = [make_rdma(my, 0, l) for l in range(3)]
        for r in p1:
            r.start()
        compute_chunk(my)
        for r in p1:
            r.wait_recv()

        p2_rdmas = [
            make_rdma(nbr[1], 1, 0),
            make_rdma(nbr[2], 1, 1),
            make_rdma(nbr[0], 1, 2),
        ]
        for r in p2_rdmas:
            r.start()
        for l in range(3):
            compute_chunk(nbr[l])
        for r in p2_rdmas:
            r.wait_recv()

        p3 = make_rdma(chunk_yz, 2, 0)
        p3.start()
        compute_chunk(chunk_xy)
        compute_chunk(chunk_xz)
        compute_chunk(chunk_yz)
        p3.wait_recv()

        compute_chunk(chunk_xyz)

        for r in p1 + p2_rdmas + [p3]:
            r.wait_send()
        copies[N_DEV - 2].wait()
        copies[N_DEV - 1].wait()

        @functools.partial(
            pl.run_scoped, second_barrier=pltpu.SemaphoreType.REGULAR
        )
        def _(second_barrier):
            for l in range(3):
                pl.semaphore_signal(
                    second_barrier, inc=1,
                    device_id=(nbr[l],), device_id_type=pl.DeviceIdType.MESH,
                )
            pl.semaphore_wait(second_barrier, 3)

    return pl.pallas_call(
        body,
        out_shape=jax.ShapeDtypeStruct((N_DEV * m_per, n), jnp.bfloat16),
        in_specs=[
            pl.BlockSpec(memory_space=pltpu.VMEM),
            pl.BlockSpec(memory_space=pltpu.VMEM),
        ],
        out_specs=pl.BlockSpec(memory_space=pltpu.MemorySpace.HBM),
        scratch_shapes=[
            pltpu.VMEM((N_DEV, m_per, k), jnp.bfloat16),
            pltpu.VMEM((2, m_per, n), jnp.bfloat16),
            pltpu.SemaphoreType.DMA((3, 3)),
            pltpu.SemaphoreType.DMA((3, 3)),
            pltpu.SemaphoreType.DMA((2,)),
        ],
        compiler_params=pltpu.CompilerParams(collective_id=0),
    )(A, B)


# device time: 109721 ns/iter; 2.4872x vs baseline; 1.1011x over previous
import jax
import jax.numpy as jnp
from jax import lax
from jax.experimental import pallas as pl
from jax.experimental.pallas import tpu as pltpu

N_DEV = 8


def kernel(A, B):
    m_per, k = A.shape
    _, n = B.shape
    third = m_per // 3

    A = A.astype(jnp.bfloat16)
    B = B.astype(jnp.bfloat16)

    def body(a_ref, b_ref, out_ref, ag_ref, stage_ref, send_sems, recv_sems,
             copy_sems):
        my = lax.axis_index("i")

        z = my // 4
        p2 = lax.rem(my, 4)
        y = p2 // 2
        x = ((p2 == 1) | (p2 == 2)).astype(my.dtype)

        def pos(xx, yy, zz):
            return zz * 4 + xx + yy * (3 - 2 * xx)

        nbr = [pos(1 - x, y, z), pos(x, 1 - y, z), pos(x, y, 1 - z)]
        chunk_xy = pos(1 - x, 1 - y, z)
        chunk_xz = pos(1 - x, y, 1 - z)
        chunk_yz = pos(x, 1 - y, 1 - z)
        chunk_xyz = pos(1 - x, 1 - y, 1 - z)

        copies = {}
        it = [0]

        def compute_chunk(chunk):
            i = it[0]
            it[0] += 1
            slot = i % 2
            if i >= 2:
                copies[i - 2].wait()
            stage_ref[slot] = jnp.dot(
                ag_ref[chunk], b_ref[...], preferred_element_type=jnp.float32
            ).astype(stage_ref.dtype)
            copies[i] = pltpu.make_async_copy(
                stage_ref.at[slot],
                out_ref.at[pl.ds(chunk * m_per, m_per)],
                copy_sems.at[slot],
            )
            copies[i].start()

        def make_rdma(chunk, phase, link, rows=None):
            src = ag_ref.at[chunk] if rows is None else ag_ref.at[chunk, rows]
            return pltpu.make_async_remote_copy(
                src_ref=src,
                dst_ref=src,
                send_sem=send_sems.at[phase, link],
                recv_sem=recv_sems.at[phase, link],
                device_id=(nbr[link],),
                device_id_type=pl.DeviceIdType.MESH,
            )

        ag_ref[my] = a_ref[...]
        compute_chunk(my)

        barrier_sem = pltpu.get_barrier_semaphore()
        for l in range(3):
            pl.semaphore_signal(
                barrier_sem, inc=1,
                device_id=(nbr[l],), device_id_type=pl.DeviceIdType.MESH,
            )
        pl.semaphore_wait(barrier_sem, 3)

        p1 = [make_rdma(my, 0, l) for l in range(3)]
        for r in p1:
            r.start()
        for r in p1:
            r.wait_recv()

        p2_rdmas = [
            make_rdma(nbr[1], 1, 0),
            make_rdma(nbr[2], 1, 1),
            make_rdma(nbr[0], 1, 2),
        ]
        for r in p2_rdmas:
            r.start()
        for l in range(3):
            compute_chunk(nbr[l])
        for r in p2_rdmas:
            r.wait_recv()

        p3 = [
            make_rdma(chunk_yz, 2, 0, rows=pl.ds(0, third)),
            make_rdma(chunk_xz, 2, 1, rows=pl.ds(third, third)),
            make_rdma(chunk_xy, 2, 2, rows=pl.ds(2 * third, third)),
        ]
        for r in p3:
            r.start()
        compute_chunk(chunk_xy)
        compute_chunk(chunk_xz)
        compute_chunk(chunk_yz)
        for r in p3:
            r.wait_recv()

        compute_chunk(chunk_xyz)

        for r in p1 + p2_rdmas + p3:
            r.wait_send()
        copies[N_DEV - 2].wait()
        copies[N_DEV - 1].wait()

    return pl.pallas_call(
        body,
        out_shape=jax.ShapeDtypeStruct((N_DEV * m_per, n), jnp.bfloat16),
        in_specs=[
            pl.BlockSpec(memory_space=pltpu.VMEM),
            pl.BlockSpec(memory_space=pltpu.VMEM),
        ],
        out_specs=pl.BlockSpec(memory_space=pltpu.MemorySpace.HBM),
        scratch_shapes=[
            pltpu.VMEM((N_DEV, m_per, k), jnp.bfloat16),
            pltpu.VMEM((2, m_per, n), jnp.bfloat16),
            pltpu.SemaphoreType.DMA((3, 3)),
            pltpu.SemaphoreType.DMA((3, 3)),
            pltpu.SemaphoreType.DMA((2,)),
        ],
        compiler_params=pltpu.CompilerParams(collective_id=0),
    )(A, B)


# device time: 105061 ns/iter; 2.5975x vs baseline; 1.0444x over previous
import jax
import jax.numpy as jnp
from jax import lax
from jax.experimental import pallas as pl
from jax.experimental.pallas import tpu as pltpu

N_DEV = 8
N_STAGE = 4


def kernel(A, B):
    m_per, k = A.shape
    _, n = B.shape
    half = m_per // 2
    third = m_per // 3

    A = A.astype(jnp.bfloat16)
    B = B.astype(jnp.bfloat16)

    def body(a_ref, b_ref, out_ref, ag_ref, stage_ref, send_sems, recv_sems,
             copy_sems):
        my = lax.axis_index("i")

        z = my // 4
        p2 = lax.rem(my, 4)
        y = p2 // 2
        x = ((p2 == 1) | (p2 == 2)).astype(my.dtype)

        def pos(xx, yy, zz):
            return zz * 4 + xx + yy * (3 - 2 * xx)

        nbr = [pos(1 - x, y, z), pos(x, 1 - y, z), pos(x, y, 1 - z)]
        chunk_xy = pos(1 - x, 1 - y, z)
        chunk_xz = pos(1 - x, y, 1 - z)
        chunk_yz = pos(x, 1 - y, 1 - z)
        chunk_xyz = pos(1 - x, 1 - y, 1 - z)

        copies = {}
        it = [0]

        def compute_half(chunk, h):
            i = it[0]
            it[0] += 1
            slot = i % N_STAGE
            if i >= N_STAGE:
                copies[i - N_STAGE].wait()
            stage_ref[slot] = jnp.dot(
                ag_ref[chunk, pl.ds(h * half, half)], b_ref[...],
                preferred_element_type=jnp.float32,
            ).astype(stage_ref.dtype)
            copies[i] = pltpu.make_async_copy(
                stage_ref.at[slot],
                out_ref.at[pl.ds(chunk * m_per + h * half, half)],
                copy_sems.at[slot],
            )
            copies[i].start()

        def make_rdma(chunk, phase, link, h, rows):
            src = ag_ref.at[chunk, rows]
            return pltpu.make_async_remote_copy(
                src_ref=src,
                dst_ref=src,
                send_sem=send_sems.at[phase, link, h],
                recv_sem=recv_sems.at[phase, link, h],
                device_id=(nbr[link],),
                device_id_type=pl.DeviceIdType.MESH,
            )

        def half_rows(h):
            return pl.ds(h * half, half)

        ag_ref[my] = a_ref[...]
        compute_half(my, 0)
        compute_half(my, 1)

        barrier_sem = pltpu.get_barrier_semaphore()
        for l in range(3):
            pl.semaphore_signal(
                barrier_sem, inc=1,
                device_id=(nbr[l],), device_id_type=pl.DeviceIdType.MESH,
            )
        pl.semaphore_wait(barrier_sem, 3)

        p1 = [[make_rdma(my, 0, l, h, half_rows(h)) for h in range(2)]
              for l in range(3)]
        for l in range(3):
            for h in range(2):
                p1[l][h].start()

        p2_src = [nbr[1], nbr[2], nbr[0]]
        p2 = [[None, None] for _ in range(3)]
        for h in range(2):
            for l in range(3):
                p1[l][h].wait_recv()
            for l in range(3):
                p2[l][h] = make_rdma(p2_src[l], 1, l, h, half_rows(h))
                p2[l][h].start()
            for l in range(3):
                compute_half(nbr[l], h)

        p3 = [None] * 3
        dist2 = [chunk_xy, chunk_yz, chunk_xz]
        for l in range(3):
            p2[l][0].wait_recv()
        p3[0] = make_rdma(chunk_yz, 2, 0, 0, pl.ds(0, third))
        p3[0].start()
        for c in dist2:
            compute_half(c, 0)
        for l in range(3):
            p2[l][1].wait_recv()
        p3[1] = make_rdma(chunk_xz, 2, 1, 0, pl.ds(third, third))
        p3[2] = make_rdma(chunk_xy, 2, 2, 0, pl.ds(2 * third, third))
        p3[1].start()
        p3[2].start()
        for c in dist2:
            compute_half(c, 1)

        for r in p3:
            r.wait_recv()
        compute_half(chunk_xyz, 0)
        compute_half(chunk_xyz, 1)

        for l in range(3):
            for h in range(2):
                p1[l][h].wait_send()
                p2[l][h].wait_send()
            p3[l].wait_send()
        for i in range(2 * N_DEV - N_STAGE, 2 * N_DEV):
            copies[i].wait()

    return pl.pallas_call(
        body,
        out_shape=jax.ShapeDtypeStruct((N_DEV * m_per, n), jnp.bfloat16),
        in_specs=[
            pl.BlockSpec(memory_space=pltpu.VMEM),
            pl.BlockSpec(memory_space=pltpu.VMEM),
        ],
        out_specs=pl.BlockSpec(memory_space=pltpu.MemorySpace.HBM),
        scratch_shapes=[
            pltpu.VMEM((N_DEV, m_per, k), jnp.bfloat16),
            pltpu.VMEM((N_STAGE, half, n), jnp.bfloat16),
            pltpu.SemaphoreType.DMA((3, 3, 2)),
            pltpu.SemaphoreType.DMA((3, 3, 2)),
            pltpu.SemaphoreType.DMA((N_STAGE,)),
        ],
        compiler_params=pltpu.CompilerParams(collective_id=0),
    )(A, B)
